# Initial kernel scaffold; baseline (speedup 1.0000x reference)
import jax
import jax.numpy as jnp
from jax.experimental import pallas as pl


def kernel(x, batch, gate_W1, gate_b1, gate_W2, gate_b2, W_ih, W_hh, b_ih, b_hh):
    raise NotImplementedError("write your pallas kernel here")


# speedup vs baseline: 5.5866x; 5.5866x over previous
import functools

import jax
import jax.numpy as jnp
from jax import lax
from jax.experimental import pallas as pl
from jax.experimental.pallas import tpu as pltpu

N = 100000
H = 128
G = 1024
B = 4000
NB = N // B
GC = 128
NGC = G // GC

_dot = functools.partial(jnp.dot, preferred_element_type=jnp.float32)


def _hilo(a):
    hi = a.astype(jnp.bfloat16)
    lo = (a - hi.astype(jnp.float32)).astype(jnp.bfloat16)
    return hi, lo


def _mm3(a, b):
    ah, al = _hilo(a)
    bh, bl = _hilo(b)
    return _dot(ah, bh) + (_dot(al, bh) + _dot(ah, bl))


def _precompute_body(x_ref, brow_ref, w1_ref, b1_ref,
                     xw1_ref, sums_ref, cnt_ref):
    i = pl.program_id(0)

    @pl.when(i == 0)
    def _():
        sums_ref[...] = jnp.zeros_like(sums_ref)
        cnt_ref[...] = jnp.zeros_like(cnt_ref)

    xb = x_ref[...]
    xw1_ref[...] = _mm3(xb, w1_ref[...]) + b1_ref[...]

    brow = brow_ref[0]
    bmin = brow_ref[0, 0, 0]
    bmax = brow_ref[0, 0, B - 1]
    xh, xl = _hilo(xb)
    for c in range(NGC):
        base = c * GC

        @pl.when((bmax >= base) & (bmin < base + GC))
        def _(base=base):
            iog = lax.broadcasted_iota(jnp.int32, (GC, B), 0) + base
            ohg = (iog == brow)
            ohb = ohg.astype(jnp.bfloat16)
            sums_ref[base:base + GC, :] += _dot(ohb, xh) + _dot(ohb, xl)
            cnt_ref[base:base + GC, :] += jnp.sum(
                ohg.astype(jnp.float32), axis=1, keepdims=True)


def _gatepass_body(xw1_ref, x_ref, brow_ref, bcol_ref, rw1_ref,
                   w2_ref, b2_ref, wsums_ref, gath_ref):
    i = pl.program_id(0)

    @pl.when(i == 0)
    def _():
        wsums_ref[...] = jnp.zeros_like(wsums_ref)

    gath_ref[...] = jnp.zeros_like(gath_ref)
    bcol = bcol_ref[...]
    brow = brow_ref[0]
    bmin = brow_ref[0, 0, 0]
    bmax = brow_ref[0, 0, B - 1]

    for c in range(NGC):
        base = c * GC

        @pl.when((bmax >= base) & (bmin < base + GC))
        def _(base=base):
            ion = lax.broadcasted_iota(jnp.int32, (B, GC), 1) + base
            ohn = (ion == bcol).astype(jnp.bfloat16)
            rh, rl = _hilo(rw1_ref[base:base + GC, :])
            gath_ref[...] += _dot(ohn, rh) + _dot(ohn, rl)

    h = jnp.maximum(xw1_ref[...] + gath_ref[...], 0.0)
    z = jnp.sum(h * w2_ref[...], axis=1, keepdims=True) + b2_ref[0, 0]
    gate = jax.nn.sigmoid(z)
    w = x_ref[...] * gate
    wh, wl = _hilo(w)
    for c in range(NGC):
        base = c * GC

        @pl.when((bmax >= base) & (bmin < base + GC))
        def _(base=base):
            iog = lax.broadcasted_iota(jnp.int32, (GC, B), 0) + base
            ohg = (iog == brow).astype(jnp.bfloat16)
            wsums_ref[base:base + GC, :] += _dot(ohg, wh) + _dot(ohg, wl)


def _meanw1_body(sums_ref, cnt_ref, w1_ref, repr_ref, rw1_ref):
    mean = sums_ref[...] / jnp.maximum(cnt_ref[...], 1.0)
    repr_ref[...] = mean
    rw1_ref[...] = _mm3(mean, w1_ref[...])


def _gru_body(wsums_ref, cnt_ref, prev_ref, wihT_ref, whhT_ref,
              bih_ref, bhh_ref, w1_ref, repr_ref, rw1_ref):
    mean = wsums_ref[...] / jnp.maximum(cnt_ref[...], 1.0)
    prev = prev_ref[...]
    gi = _mm3(mean, wihT_ref[...]) + bih_ref[...]
    gh = _mm3(prev, whhT_ref[...]) + bhh_ref[...]
    r = jax.nn.sigmoid(gi[:, :H] + gh[:, :H])
    z = jax.nn.sigmoid(gi[:, H:2 * H] + gh[:, H:2 * H])
    n = jnp.tanh(gi[:, 2 * H:] + r * gh[:, 2 * H:])
    new = jnp.maximum((1.0 - z) * n + z * prev, 0.0)
    repr_ref[...] = new
    rw1_ref[...] = _mm3(new, w1_ref[...])


def kernel(x, batch, gate_W1, gate_b1, gate_W2, gate_b2,
           W_ih, W_hh, b_ih, b_hh):
    batch = batch.astype(jnp.int32)
    brow = batch.reshape(NB, 1, B)
    bcol = batch.reshape(N, 1)
    b1r = gate_b1.reshape(1, H)
    w2r = gate_W2.reshape(1, H)
    b2p = jnp.broadcast_to(gate_b2.reshape(1, 1), (1, GC))
    wihT = W_ih.T
    whhT = W_hh.T
    bihr = b_ih.reshape(1, 3 * H)
    bhhr = b_hh.reshape(1, 3 * H)

    f32 = jnp.float32
    const = lambda shape: pl.BlockSpec(shape, lambda i: tuple(0 for _ in shape))

    xw1, sums0, cnt = pl.pallas_call(
        _precompute_body,
        grid=(NB,),
        in_specs=[
            pl.BlockSpec((B, H), lambda i: (i, 0)),
            pl.BlockSpec((1, 1, B), lambda i: (i, 0, 0)),
            const((H, H)),
            const((1, H)),
        ],
        out_specs=[
            pl.BlockSpec((B, H), lambda i: (i, 0)),
            const((G, H)),
            const((G, 1)),
        ],
        out_shape=[
            jax.ShapeDtypeStruct((N, H), f32),
            jax.ShapeDtypeStruct((G, H), f32),
            jax.ShapeDtypeStruct((G, 1), f32),
        ],
    )(x, brow, gate_W1, b1r)

    small = dict(grid=(1,))
    repr_, rw1 = pl.pallas_call(
        _meanw1_body,
        in_specs=[const((G, H)), const((G, 1)), const((H, H))],
        out_specs=[const((G, H)), const((G, H))],
        out_shape=[jax.ShapeDtypeStruct((G, H), f32),
                   jax.ShapeDtypeStruct((G, H), f32)],
        **small,
    )(sums0, cnt, gate_W1)

    gatepass = pl.pallas_call(
        _gatepass_body,
        grid=(NB,),
        in_specs=[
            pl.BlockSpec((B, H), lambda i: (i, 0)),
            pl.BlockSpec((B, H), lambda i: (i, 0)),
            pl.BlockSpec((1, 1, B), lambda i: (i, 0, 0)),
            pl.BlockSpec((B, 1), lambda i: (i, 0)),
            const((G, H)),
            const((1, H)),
            const((1, GC)),
        ],
        out_specs=[const((G, H))],
        out_shape=[jax.ShapeDtypeStruct((G, H), f32)],
        scratch_shapes=[pltpu.VMEM((B, H), f32)],
    )

    gru = pl.pallas_call(
        _gru_body,
        in_specs=[const((G, H)), const((G, 1)), const((G, H)),
                  const((H, 3 * H)), const((H, 3 * H)),
                  const((1, 3 * H)), const((1, 3 * H)), const((H, H))],
        out_specs=[const((G, H)), const((G, H))],
        out_shape=[jax.ShapeDtypeStruct((G, H), f32),
                   jax.ShapeDtypeStruct((G, H), f32)],
        **small,
    )

    for _ in range(2):
        (wsums,) = gatepass(xw1, x, brow, bcol, rw1, w2r, b2p)
        repr_, rw1 = gru(wsums, cnt, repr_, wihT, whhT, bihr, bhhr, gate_W1)

    return repr_
